# baseline (speedup 1.0000x reference)
import jax
import jax.numpy as jnp
from jax.experimental import pallas as pl


def _add_body(tok_ref, pos_ref, out_ref):
    out_ref[...] = tok_ref[...] + pos_ref[...]


def kernel(token_embedding, src_mask, pos_embedding):
    B, S, E = token_embedding.shape
    BS = 512
    nsb = S // BS
    tok2d = token_embedding.reshape(B * S, E)
    out2d = pl.pallas_call(
        _add_body,
        grid=(B * S // BS,),
        in_specs=[
            pl.BlockSpec((BS, E), lambda i: (i, 0)),
            pl.BlockSpec((BS, E), lambda i: (jax.lax.rem(i, nsb), 0)),
        ],
        out_specs=pl.BlockSpec((BS, E), lambda i: (i, 0)),
        out_shape=jax.ShapeDtypeStruct((B * S, E), token_embedding.dtype),
    )(tok2d, pos_embedding[:S])
    return out2d.reshape(B, S, E)
